# baseline (speedup 1.0000x reference)
import functools

import jax
import jax.numpy as jnp
from jax import lax
from jax.experimental import pallas as pl
from jax.experimental.pallas import tpu as pltpu
from jax.experimental.pallas import tpu_sc as plsc

N = 10000
E = 320000
D = 128
NC = 2
NS = 16
NW = NC * NS
EPW = E // NW
K = 80
NCHUNK = 125
EPWP = NCHUNK * K
TROWS = 640
ACC_ROWS = NS * TROWS
DUMMY = N
ZROWS = 40
BM = 10000
GRID = N // BM

_mesh = lambda: plsc.VectorSubcoreMesh(core_axis_name="c", subcore_axis_name="s")
_SC_PARAMS = pltpu.CompilerParams(use_tc_tiling_on_sc=False)


def _sc_prep(src3, dst3, ones_h, z16_h):
    @functools.partial(
        pl.kernel,
        out_type=(
            jax.ShapeDtypeStruct((NW, NCHUNK, K), jnp.int32),
            jax.ShapeDtypeStruct((NC * ACC_ROWS, 16), jnp.float32),
        ),
        mesh=_mesh(),
        scratch_types=(
            pltpu.VMEM((NCHUNK, K), jnp.int32),
            pltpu.VMEM((NCHUNK, K), jnp.int32),
            pltpu.VMEM((NCHUNK, K), jnp.int32),
            pltpu.VMEM((NCHUNK, K), jnp.int32),
            pltpu.VMEM((K, 16), jnp.float32),
            pltpu.VMEM((TROWS, 16), jnp.float32),
            pltpu.SemaphoreType.DMA,
            pltpu.VMEM_SHARED((ACC_ROWS, 16), jnp.float32),
        ),
        compiler_params=_SC_PARAMS,
    )
    def body(src3_h, dst3_h, ones_hbm, z16_hbm, dstm3_o, degp_o,
             src_v, dst_v, srcm_v, dstm_v, ones_v, stage_v, sh, deg_sh):
        cid = lax.axis_index("c")
        sid = lax.axis_index("s")
        wid = sid * NC + cid
        pltpu.sync_copy(src3_h.at[wid], src_v)
        pltpu.sync_copy(dst3_h.at[wid], dst_v)
        pltpu.sync_copy(ones_hbm, ones_v)

        def mask_row(c, carry):
            for j in range(K // 16):
                sl = pl.ds(j * 16, 16)
                s = src_v[c, sl]
                d = dst_v[c, sl]
                eq = s == d
                srcm_v[c, sl] = jnp.where(eq, DUMMY, jnp.where(d >= N, d, s))
                dstm_v[c, sl] = jnp.where(eq, DUMMY, d)
            return carry

        lax.fori_loop(0, NCHUNK, mask_row, 0)
        pltpu.sync_copy(dstm_v, dstm3_o.at[wid])
        pltpu.sync_copy(z16_hbm, stage_v)
        pltpu.sync_copy(stage_v, deg_sh.at[pl.ds(sid * TROWS, TROWS)])
        plsc.subcore_barrier()

        def hist_fire(c, carry):
            pltpu.async_copy(ones_v, deg_sh.at[srcm_v.at[c]], sh, add=True)
            return carry

        lax.fori_loop(0, NCHUNK, hist_fire, 0)

        def hist_drain(c, carry):
            pltpu.make_async_copy(
                ones_v, deg_sh.at[srcm_v.at[c]], sh).wait()
            return carry

        lax.fori_loop(0, NCHUNK, hist_drain, 0)
        plsc.subcore_barrier()
        pltpu.sync_copy(deg_sh.at[pl.ds(sid * TROWS, TROWS)], stage_v)
        pltpu.sync_copy(
            stage_v,
            degp_o.at[pl.ds(cid * ACC_ROWS + sid * TROWS, TROWS)])

    return body(src3, dst3, ones_h, z16_h)


def _sc_agg(g, src3, dstm3, z128_h):
    @functools.partial(
        pl.kernel,
        out_type=jax.ShapeDtypeStruct((NC * ACC_ROWS, D), jnp.float32),
        mesh=_mesh(),
        scratch_types=(
            pltpu.VMEM((NCHUNK, K), jnp.int32),
            pltpu.VMEM((NCHUNK, K), jnp.int32),
            pltpu.VMEM((K, D), jnp.float32),
            pltpu.VMEM((K, D), jnp.float32),
            pltpu.VMEM((ZROWS, D), jnp.float32),
            pltpu.SemaphoreType.DMA,
            pltpu.SemaphoreType.DMA,
            pltpu.SemaphoreType.DMA,
            pltpu.VMEM_SHARED((ACC_ROWS, D), jnp.float32),
        ),
        compiler_params=_SC_PARAMS,
    )
    def body(g_h, src3_h, dstm3_h, z128_hbm, accp_o,
             src_v, dstm_v, rows_a, rows_b, stage_v, sra, srb, srz, acc_sh):
        cid = lax.axis_index("c")
        sid = lax.axis_index("s")
        wid = sid * NC + cid
        pltpu.sync_copy(src3_h.at[wid], src_v)
        pltpu.async_copy(g_h.at[src_v.at[0]], rows_a, sra)
        pltpu.async_copy(g_h.at[src_v.at[1]], rows_b, srb)
        pltpu.sync_copy(dstm3_h.at[wid], dstm_v)
        pltpu.sync_copy(z128_hbm, stage_v)
        for p in range(TROWS // ZROWS):
            pltpu.async_copy(
                stage_v, acc_sh.at[pl.ds(sid * TROWS + p * ZROWS, ZROWS)],
                srz)
        for p in range(TROWS // ZROWS):
            pltpu.make_async_copy(
                stage_v, acc_sh.at[pl.ds(sid * TROWS + p * ZROWS, ZROWS)],
                srz).wait()
        plsc.subcore_barrier()

        def pair(i, carry):
            c0 = 2 * i
            c1 = c0 + 1
            pltpu.make_async_copy(g_h.at[src_v.at[c0]], rows_a, sra).wait()
            pltpu.sync_copy(rows_a, acc_sh.at[dstm_v.at[c0]], add=True)
            pltpu.async_copy(g_h.at[src_v.at[c0 + 2]], rows_a, sra)
            pltpu.make_async_copy(g_h.at[src_v.at[c1]], rows_b, srb).wait()
            pltpu.sync_copy(rows_b, acc_sh.at[dstm_v.at[c1]], add=True)

            @pl.when(c1 + 2 < NCHUNK)
            def _():
                pltpu.async_copy(g_h.at[src_v.at[c1 + 2]], rows_b, srb)

            return carry

        lax.fori_loop(0, (NCHUNK - 1) // 2, pair, 0)
        pltpu.make_async_copy(
            g_h.at[src_v.at[NCHUNK - 1]], rows_a, sra).wait()
        pltpu.sync_copy(rows_a, acc_sh.at[dstm_v.at[NCHUNK - 1]], add=True)
        plsc.subcore_barrier()
        nwb = TROWS // K
        for p in range(nwb):
            buf, sem = (rows_a, sra) if p % 2 == 0 else (rows_b, srb)
            if p >= 2:
                pltpu.make_async_copy(
                    buf,
                    accp_o.at[pl.ds(
                        cid * ACC_ROWS + sid * TROWS + (p - 2) * K, K)],
                    sem).wait()
            pltpu.sync_copy(acc_sh.at[pl.ds(sid * TROWS + p * K, K)], buf)
            pltpu.async_copy(
                buf,
                accp_o.at[pl.ds(cid * ACC_ROWS + sid * TROWS + p * K, K)],
                sem)
        for p in range(nwb - 2, nwb):
            buf, sem = (rows_a, sra) if p % 2 == 0 else (rows_b, srb)
            pltpu.make_async_copy(
                buf,
                accp_o.at[pl.ds(cid * ACC_ROWS + sid * TROWS + p * K, K)],
                sem).wait()

    return body(g, src3, dstm3, z128_h)


def _dis_of(degp_ref):
    deg = (jnp.mean(degp_ref[0], axis=1) + jnp.mean(degp_ref[1], axis=1)
           + jnp.float32(1.0))
    return lax.rsqrt(deg)


def _tc_g1_body(degp_ref, x_ref, w_ref, out_ref):
    dis = _dis_of(degp_ref)
    h = jnp.dot(x_ref[...], w_ref[...], preferred_element_type=jnp.float32)
    out_ref[...] = dis[:, None] * h


def _tc_mid_body(degp_ref, acc_ref, g1_ref, w_ref, b_ref, out_ref):
    dis = _dis_of(degp_ref)
    s = acc_ref[0] + acc_ref[1] + g1_ref[...]
    t = jnp.maximum(dis[:, None] * s + b_ref[...], jnp.float32(0.0))
    out_ref[...] = dis[:, None] * jnp.dot(
        t, w_ref[...], preferred_element_type=jnp.float32)


def _tc_fin_body(degp_ref, acc_ref, g2_ref, b_ref, out_ref):
    dis = _dis_of(degp_ref)
    s = acc_ref[0] + acc_ref[1] + g2_ref[...]
    out_ref[...] = dis[:, None] * s + b_ref[...]


_DEG_SPEC = pl.BlockSpec((NC, BM, 16), lambda i: (0, i, 0))
_ROW_SPEC = pl.BlockSpec((BM, D), lambda i: (i, 0))
_ACC_SPEC = pl.BlockSpec((NC, BM, D), lambda i: (0, i, 0))
_W_SPEC = pl.BlockSpec((D, D), lambda i: (0, 0))
_B_SPEC = pl.BlockSpec((1, D), lambda i: (0, 0))
_OUT = jax.ShapeDtypeStruct((N, D), jnp.float32)

_tc_g1 = pl.pallas_call(
    _tc_g1_body, grid=(GRID,),
    in_specs=[_DEG_SPEC, _ROW_SPEC, _W_SPEC],
    out_specs=_ROW_SPEC, out_shape=_OUT)

_tc_mid = pl.pallas_call(
    _tc_mid_body, grid=(GRID,),
    in_specs=[_DEG_SPEC, _ACC_SPEC, _ROW_SPEC, _W_SPEC, _B_SPEC],
    out_specs=_ROW_SPEC, out_shape=_OUT)

_tc_fin = pl.pallas_call(
    _tc_fin_body, grid=(GRID,),
    in_specs=[_DEG_SPEC, _ACC_SPEC, _ROW_SPEC, _B_SPEC],
    out_specs=_ROW_SPEC, out_shape=_OUT)


def _pad_edges(v, pad_row):
    pad = jnp.broadcast_to(pad_row, (NW, EPWP - EPW))
    return jnp.concatenate(
        [v.astype(jnp.int32).reshape(NW, EPW), pad], axis=1
    ).reshape(NW, NCHUNK, K)


def kernel(x, edge_index, W1, b1, W2, b2):
    src3 = edge_index[0].astype(jnp.int32).reshape(NW, NCHUNK, K)
    dst3 = edge_index[1].astype(jnp.int32).reshape(NW, NCHUNK, K)
    ones16 = jnp.ones((K, 16), jnp.float32)
    z16 = jnp.zeros((TROWS, 16), jnp.float32)
    z128 = jnp.zeros((ZROWS, D), jnp.float32)

    dstm3, degp = _sc_prep(src3, dst3, ones16, z16)
    degp3 = degp.reshape(NC, ACC_ROWS, 16)

    g1 = _tc_g1(degp3, x, W1)
    acc1 = _sc_agg(g1, src3, dstm3, z128).reshape(NC, ACC_ROWS, D)
    g2 = _tc_mid(degp3, acc1, g1, W2, b1.reshape(1, D))
    acc2 = _sc_agg(g2, src3, dstm3, z128).reshape(NC, ACC_ROWS, D)
    return _tc_fin(degp3, acc2, g2, b2.reshape(1, D))


# speedup vs baseline: 1.0116x; 1.0116x over previous
import functools

import jax
import jax.numpy as jnp
from jax import lax
from jax.experimental import pallas as pl
from jax.experimental.pallas import tpu as pltpu
from jax.experimental.pallas import tpu_sc as plsc

N = 10000
E = 320000
D = 128
NC = 2
NS = 16
NW = NC * NS
EPW = E // NW
K = 80
NCHUNK = 125
EPWP = NCHUNK * K
TROWS = 640
ACC_ROWS = NS * TROWS
DUMMY = N
ZROWS = 40
BM = 2000
GRID = N // BM

_mesh = lambda: plsc.VectorSubcoreMesh(core_axis_name="c", subcore_axis_name="s")
_SC_PARAMS = pltpu.CompilerParams(use_tc_tiling_on_sc=False)


def _sc_prep(src3, dst3, ones_h, z16_h):
    @functools.partial(
        pl.kernel,
        out_type=(
            jax.ShapeDtypeStruct((NW, NCHUNK, K), jnp.int32),
            jax.ShapeDtypeStruct((NC * ACC_ROWS, 16), jnp.float32),
        ),
        mesh=_mesh(),
        scratch_types=(
            pltpu.VMEM((NCHUNK, K), jnp.int32),
            pltpu.VMEM((NCHUNK, K), jnp.int32),
            pltpu.VMEM((NCHUNK, K), jnp.int32),
            pltpu.VMEM((NCHUNK, K), jnp.int32),
            pltpu.VMEM((K, 16), jnp.float32),
            pltpu.VMEM((TROWS, 16), jnp.float32),
            pltpu.SemaphoreType.DMA,
            pltpu.SemaphoreType.DMA,
            pltpu.VMEM_SHARED((ACC_ROWS, 16), jnp.float32),
        ),
        compiler_params=_SC_PARAMS,
    )
    def body(src3_h, dst3_h, ones_hbm, z16_hbm, dstm3_o, degp_o,
             src_v, dst_v, srcm_v, dstm_v, ones_v, stage_v, sh, sw, deg_sh):
        cid = lax.axis_index("c")
        sid = lax.axis_index("s")
        wid = sid * NC + cid
        pltpu.sync_copy(src3_h.at[wid], src_v)
        pltpu.sync_copy(dst3_h.at[wid], dst_v)
        pltpu.sync_copy(ones_hbm, ones_v)
        pltpu.sync_copy(z16_hbm, stage_v)
        pltpu.async_copy(stage_v, deg_sh.at[pl.ds(sid * TROWS, TROWS)], sw)

        def mask_row(c, carry):
            for j in range(K // 16):
                sl = pl.ds(j * 16, 16)
                s = src_v[c, sl]
                d = dst_v[c, sl]
                eq = s == d
                srcm_v[c, sl] = jnp.where(eq, DUMMY, jnp.where(d >= N, d, s))
                dstm_v[c, sl] = jnp.where(eq, DUMMY, d)
            return carry

        lax.fori_loop(0, NCHUNK, mask_row, 0)
        pltpu.make_async_copy(
            stage_v, deg_sh.at[pl.ds(sid * TROWS, TROWS)], sw).wait()
        pltpu.async_copy(dstm_v, dstm3_o.at[wid], sw)
        plsc.subcore_barrier()

        def hist_fire(c, carry):
            pltpu.async_copy(ones_v, deg_sh.at[srcm_v.at[c]], sh, add=True)
            return carry

        lax.fori_loop(0, NCHUNK, hist_fire, 0)

        def hist_drain(c, carry):
            pltpu.make_async_copy(
                ones_v, deg_sh.at[srcm_v.at[c]], sh).wait()
            return carry

        lax.fori_loop(0, NCHUNK, hist_drain, 0)
        plsc.subcore_barrier()
        pltpu.sync_copy(deg_sh.at[pl.ds(sid * TROWS, TROWS)], stage_v)
        pltpu.sync_copy(
            stage_v,
            degp_o.at[pl.ds(cid * ACC_ROWS + sid * TROWS, TROWS)])
        pltpu.make_async_copy(dstm_v, dstm3_o.at[wid], sw).wait()

    return body(src3, dst3, ones_h, z16_h)


def _sc_agg(g, src3, dstm3, z128_h):
    @functools.partial(
        pl.kernel,
        out_type=jax.ShapeDtypeStruct((NC * ACC_ROWS, D), jnp.float32),
        mesh=_mesh(),
        scratch_types=(
            pltpu.VMEM((NCHUNK, K), jnp.int32),
            pltpu.VMEM((NCHUNK, K), jnp.int32),
            pltpu.VMEM((K, D), jnp.float32),
            pltpu.VMEM((K, D), jnp.float32),
            pltpu.VMEM((ZROWS, D), jnp.float32),
            pltpu.SemaphoreType.DMA,
            pltpu.SemaphoreType.DMA,
            pltpu.SemaphoreType.DMA,
            pltpu.VMEM_SHARED((ACC_ROWS, D), jnp.float32),
        ),
        compiler_params=_SC_PARAMS,
    )
    def body(g_h, src3_h, dstm3_h, z128_hbm, accp_o,
             src_v, dstm_v, rows_a, rows_b, stage_v, sra, srb, srz, acc_sh):
        cid = lax.axis_index("c")
        sid = lax.axis_index("s")
        wid = sid * NC + cid
        pltpu.sync_copy(src3_h.at[wid], src_v)
        pltpu.async_copy(g_h.at[src_v.at[0]], rows_a, sra)
        pltpu.async_copy(g_h.at[src_v.at[1]], rows_b, srb)
        pltpu.sync_copy(dstm3_h.at[wid], dstm_v)
        pltpu.sync_copy(z128_hbm, stage_v)
        for p in range(TROWS // ZROWS):
            pltpu.async_copy(
                stage_v, acc_sh.at[pl.ds(sid * TROWS + p * ZROWS, ZROWS)],
                srz)
        for p in range(TROWS // ZROWS):
            pltpu.make_async_copy(
                stage_v, acc_sh.at[pl.ds(sid * TROWS + p * ZROWS, ZROWS)],
                srz).wait()
        plsc.subcore_barrier()

        def pair(i, carry):
            c0 = 2 * i
            c1 = c0 + 1
            pltpu.make_async_copy(g_h.at[src_v.at[c0]], rows_a, sra).wait()
            pltpu.sync_copy(rows_a, acc_sh.at[dstm_v.at[c0]], add=True)
            pltpu.async_copy(g_h.at[src_v.at[c0 + 2]], rows_a, sra)
            pltpu.make_async_copy(g_h.at[src_v.at[c1]], rows_b, srb).wait()
            pltpu.sync_copy(rows_b, acc_sh.at[dstm_v.at[c1]], add=True)

            @pl.when(c1 + 2 < NCHUNK)
            def _():
                pltpu.async_copy(g_h.at[src_v.at[c1 + 2]], rows_b, srb)

            return carry

        lax.fori_loop(0, (NCHUNK - 1) // 2, pair, 0)
        pltpu.make_async_copy(
            g_h.at[src_v.at[NCHUNK - 1]], rows_a, sra).wait()
        pltpu.sync_copy(rows_a, acc_sh.at[dstm_v.at[NCHUNK - 1]], add=True)
        plsc.subcore_barrier()
        nwb = TROWS // K
        for p in range(nwb):
            buf, sem = (rows_a, sra) if p % 2 == 0 else (rows_b, srb)
            if p >= 2:
                pltpu.make_async_copy(
                    buf,
                    accp_o.at[pl.ds(
                        cid * ACC_ROWS + sid * TROWS + (p - 2) * K, K)],
                    sem).wait()
            pltpu.sync_copy(acc_sh.at[pl.ds(sid * TROWS + p * K, K)], buf)
            pltpu.async_copy(
                buf,
                accp_o.at[pl.ds(cid * ACC_ROWS + sid * TROWS + p * K, K)],
                sem)
        for p in range(nwb - 2, nwb):
            buf, sem = (rows_a, sra) if p % 2 == 0 else (rows_b, srb)
            pltpu.make_async_copy(
                buf,
                accp_o.at[pl.ds(cid * ACC_ROWS + sid * TROWS + p * K, K)],
                sem).wait()

    return body(g, src3, dstm3, z128_h)


def _dis_of(degp_ref):
    deg = (jnp.mean(degp_ref[0], axis=1) + jnp.mean(degp_ref[1], axis=1)
           + jnp.float32(1.0))
    return lax.rsqrt(deg)


def _tc_g1_body(degp_ref, x_ref, w_ref, out_ref):
    dis = _dis_of(degp_ref)
    h = jnp.dot(x_ref[...], w_ref[...], preferred_element_type=jnp.float32)
    out_ref[...] = dis[:, None] * h


def _tc_mid_body(degp_ref, acc_ref, g1_ref, w_ref, b_ref, out_ref):
    dis = _dis_of(degp_ref)
    s = acc_ref[0] + acc_ref[1] + g1_ref[...]
    t = jnp.maximum(dis[:, None] * s + b_ref[...], jnp.float32(0.0))
    out_ref[...] = dis[:, None] * jnp.dot(
        t, w_ref[...], preferred_element_type=jnp.float32)


def _tc_fin_body(degp_ref, acc_ref, g2_ref, b_ref, out_ref):
    dis = _dis_of(degp_ref)
    s = acc_ref[0] + acc_ref[1] + g2_ref[...]
    out_ref[...] = dis[:, None] * s + b_ref[...]


_DEG_SPEC = pl.BlockSpec((NC, BM, 16), lambda i: (0, i, 0))
_ROW_SPEC = pl.BlockSpec((BM, D), lambda i: (i, 0))
_ACC_SPEC = pl.BlockSpec((NC, BM, D), lambda i: (0, i, 0))
_W_SPEC = pl.BlockSpec((D, D), lambda i: (0, 0))
_B_SPEC = pl.BlockSpec((1, D), lambda i: (0, 0))
_OUT = jax.ShapeDtypeStruct((N, D), jnp.float32)

_tc_g1 = pl.pallas_call(
    _tc_g1_body, grid=(GRID,),
    in_specs=[_DEG_SPEC, _ROW_SPEC, _W_SPEC],
    out_specs=_ROW_SPEC, out_shape=_OUT)

_tc_mid = pl.pallas_call(
    _tc_mid_body, grid=(GRID,),
    in_specs=[_DEG_SPEC, _ACC_SPEC, _ROW_SPEC, _W_SPEC, _B_SPEC],
    out_specs=_ROW_SPEC, out_shape=_OUT)

_tc_fin = pl.pallas_call(
    _tc_fin_body, grid=(GRID,),
    in_specs=[_DEG_SPEC, _ACC_SPEC, _ROW_SPEC, _B_SPEC],
    out_specs=_ROW_SPEC, out_shape=_OUT)


def _pad_edges(v, pad_row):
    pad = jnp.broadcast_to(pad_row, (NW, EPWP - EPW))
    return jnp.concatenate(
        [v.astype(jnp.int32).reshape(NW, EPW), pad], axis=1
    ).reshape(NW, NCHUNK, K)


def kernel(x, edge_index, W1, b1, W2, b2):
    src3 = edge_index[0].astype(jnp.int32).reshape(NW, NCHUNK, K)
    dst3 = edge_index[1].astype(jnp.int32).reshape(NW, NCHUNK, K)
    ones16 = jnp.ones((K, 16), jnp.float32)
    z16 = jnp.zeros((TROWS, 16), jnp.float32)
    z128 = jnp.zeros((ZROWS, D), jnp.float32)

    dstm3, degp = _sc_prep(src3, dst3, ones16, z16)
    degp3 = degp.reshape(NC, ACC_ROWS, 16)

    g1 = _tc_g1(degp3, x, W1)
    acc1 = _sc_agg(g1, src3, dstm3, z128).reshape(NC, ACC_ROWS, D)
    g2 = _tc_mid(degp3, acc1, g1, W2, b1.reshape(1, D))
    acc2 = _sc_agg(g2, src3, dstm3, z128).reshape(NC, ACC_ROWS, D)
    return _tc_fin(degp3, acc2, g2, b2.reshape(1, D))


# speedup vs baseline: 1.0120x; 1.0004x over previous
import functools

import jax
import jax.numpy as jnp
from jax import lax
from jax.experimental import pallas as pl
from jax.experimental.pallas import tpu as pltpu
from jax.experimental.pallas import tpu_sc as plsc

N = 10000
E = 320000
D = 128
NC = 2
NS = 16
NW = NC * NS
EPW = E // NW
K = 80
NCHUNK = 125
TROWS = 640
ACC_ROWS = NS * TROWS
DUMMY = N
ZROWS = 40
BM = 2000
GRID = N // BM

_mesh = lambda: plsc.VectorSubcoreMesh(core_axis_name="c", subcore_axis_name="s")
_SC_PARAMS = pltpu.CompilerParams(use_tc_tiling_on_sc=False)


def _sc_prep(src3, dst3, ones_h, z16_h):
    @functools.partial(
        pl.kernel,
        out_type=(
            jax.ShapeDtypeStruct((NW, NCHUNK, K), jnp.int32),
            jax.ShapeDtypeStruct((NC * ACC_ROWS, 16), jnp.float32),
        ),
        mesh=_mesh(),
        scratch_types=(
            pltpu.VMEM((NCHUNK, K), jnp.int32),
            pltpu.VMEM((NCHUNK, K), jnp.int32),
            pltpu.VMEM((NCHUNK, K), jnp.int32),
            pltpu.VMEM((NCHUNK, K), jnp.int32),
            pltpu.VMEM((K, 16), jnp.float32),
            pltpu.VMEM((TROWS, 16), jnp.float32),
            pltpu.SemaphoreType.DMA,
            pltpu.SemaphoreType.DMA,
            pltpu.VMEM_SHARED((ACC_ROWS, 16), jnp.float32),
        ),
        compiler_params=_SC_PARAMS,
    )
    def body(src3_h, dst3_h, ones_hbm, z16_hbm, dstm3_o, degp_o,
             src_v, dst_v, srcm_v, dstm_v, ones_v, stage_v, sh, sw, deg_sh):
        cid = lax.axis_index("c")
        sid = lax.axis_index("s")
        wid = sid * NC + cid
        pltpu.sync_copy(src3_h.at[wid], src_v)
        pltpu.sync_copy(dst3_h.at[wid], dst_v)
        pltpu.sync_copy(ones_hbm, ones_v)
        pltpu.sync_copy(z16_hbm, stage_v)
        pltpu.async_copy(stage_v, deg_sh.at[pl.ds(sid * TROWS, TROWS)], sw)

        def mask_row(c, carry):
            for j in range(K // 16):
                sl = pl.ds(j * 16, 16)
                s = src_v[c, sl]
                d = dst_v[c, sl]
                eq = s == d
                srcm_v[c, sl] = jnp.where(eq, DUMMY, s)
                dstm_v[c, sl] = jnp.where(eq, DUMMY, d)
            return carry

        lax.fori_loop(0, NCHUNK, mask_row, 0)
        pltpu.make_async_copy(
            stage_v, deg_sh.at[pl.ds(sid * TROWS, TROWS)], sw).wait()
        pltpu.async_copy(dstm_v, dstm3_o.at[wid], sw)
        plsc.subcore_barrier()

        def hist_fire(c, carry):
            pltpu.async_copy(ones_v, deg_sh.at[srcm_v.at[c]], sh, add=True)
            return carry

        lax.fori_loop(0, NCHUNK, hist_fire, 0)

        def hist_drain(c, carry):
            pltpu.make_async_copy(
                ones_v, deg_sh.at[srcm_v.at[c]], sh).wait()
            return carry

        lax.fori_loop(0, NCHUNK, hist_drain, 0)
        plsc.subcore_barrier()
        pltpu.sync_copy(deg_sh.at[pl.ds(sid * TROWS, TROWS)], stage_v)
        pltpu.sync_copy(
            stage_v,
            degp_o.at[pl.ds(cid * ACC_ROWS + sid * TROWS, TROWS)])
        pltpu.make_async_copy(dstm_v, dstm3_o.at[wid], sw).wait()

    return body(src3, dst3, ones_h, z16_h)


def _sc_agg(g, src3, dstm3, z128_h):
    @functools.partial(
        pl.kernel,
        out_type=jax.ShapeDtypeStruct((NC * ACC_ROWS, D), jnp.float32),
        mesh=_mesh(),
        scratch_types=(
            pltpu.VMEM((NCHUNK, K), jnp.int32),
            pltpu.VMEM((NCHUNK, K), jnp.int32),
            pltpu.VMEM((K, D), jnp.float32),
            pltpu.VMEM((K, D), jnp.float32),
            pltpu.VMEM((ZROWS, D), jnp.float32),
            pltpu.SemaphoreType.DMA,
            pltpu.SemaphoreType.DMA,
            pltpu.SemaphoreType.DMA,
            pltpu.VMEM_SHARED((ACC_ROWS, D), jnp.float32),
        ),
        compiler_params=_SC_PARAMS,
    )
    def body(g_h, src3_h, dstm3_h, z128_hbm, accp_o,
             src_v, dstm_v, rows_a, rows_b, stage_v, sra, srb, srz, acc_sh):
        cid = lax.axis_index("c")
        sid = lax.axis_index("s")
        wid = sid * NC + cid
        pltpu.sync_copy(src3_h.at[wid], src_v)
        pltpu.async_copy(g_h.at[src_v.at[0]], rows_a, sra)
        pltpu.async_copy(g_h.at[src_v.at[1]], rows_b, srb)
        pltpu.sync_copy(dstm3_h.at[wid], dstm_v)
        pltpu.sync_copy(z128_hbm, stage_v)
        for p in range(TROWS // ZROWS):
            pltpu.async_copy(
                stage_v, acc_sh.at[pl.ds(sid * TROWS + p * ZROWS, ZROWS)],
                srz)
        for p in range(TROWS // ZROWS):
            pltpu.make_async_copy(
                stage_v, acc_sh.at[pl.ds(sid * TROWS + p * ZROWS, ZROWS)],
                srz).wait()
        plsc.subcore_barrier()

        def pair(i, carry):
            c0 = 2 * i
            c1 = c0 + 1
            pltpu.make_async_copy(g_h.at[src_v.at[c0]], rows_a, sra).wait()
            pltpu.sync_copy(rows_a, acc_sh.at[dstm_v.at[c0]], add=True)
            pltpu.async_copy(g_h.at[src_v.at[c0 + 2]], rows_a, sra)
            pltpu.make_async_copy(g_h.at[src_v.at[c1]], rows_b, srb).wait()
            pltpu.sync_copy(rows_b, acc_sh.at[dstm_v.at[c1]], add=True)

            @pl.when(c1 + 2 < NCHUNK)
            def _():
                pltpu.async_copy(g_h.at[src_v.at[c1 + 2]], rows_b, srb)

            return carry

        lax.fori_loop(0, (NCHUNK - 1) // 2, pair, 0)
        pltpu.make_async_copy(
            g_h.at[src_v.at[NCHUNK - 1]], rows_a, sra).wait()
        pltpu.sync_copy(rows_a, acc_sh.at[dstm_v.at[NCHUNK - 1]], add=True)
        plsc.subcore_barrier()
        nwb = TROWS // K
        for p in range(nwb):
            buf, sem = (rows_a, sra) if p % 2 == 0 else (rows_b, srb)
            if p >= 2:
                pltpu.make_async_copy(
                    buf,
                    accp_o.at[pl.ds(
                        cid * ACC_ROWS + sid * TROWS + (p - 2) * K, K)],
                    sem).wait()
            pltpu.sync_copy(acc_sh.at[pl.ds(sid * TROWS + p * K, K)], buf)
            pltpu.async_copy(
                buf,
                accp_o.at[pl.ds(cid * ACC_ROWS + sid * TROWS + p * K, K)],
                sem)
        for p in range(nwb - 2, nwb):
            buf, sem = (rows_a, sra) if p % 2 == 0 else (rows_b, srb)
            pltpu.make_async_copy(
                buf,
                accp_o.at[pl.ds(cid * ACC_ROWS + sid * TROWS + p * K, K)],
                sem).wait()

    return body(g, src3, dstm3, z128_h)


def _dis_of(degp_ref):
    deg = (jnp.mean(degp_ref[0], axis=1) + jnp.mean(degp_ref[1], axis=1)
           + jnp.float32(1.0))
    return lax.rsqrt(deg)


def _tc_g1_body(degp_ref, x_ref, w_ref, out_ref):
    dis = _dis_of(degp_ref)
    h = jnp.dot(x_ref[...], w_ref[...], preferred_element_type=jnp.float32)
    out_ref[...] = dis[:, None] * h


def _tc_mid_body(degp_ref, acc_ref, g1_ref, w_ref, b_ref, out_ref):
    dis = _dis_of(degp_ref)
    s = acc_ref[0] + acc_ref[1] + g1_ref[...]
    t = jnp.maximum(dis[:, None] * s + b_ref[...], jnp.float32(0.0))
    out_ref[...] = dis[:, None] * jnp.dot(
        t, w_ref[...], preferred_element_type=jnp.float32)


def _tc_fin_body(degp_ref, acc_ref, g2_ref, b_ref, out_ref):
    dis = _dis_of(degp_ref)
    s = acc_ref[0] + acc_ref[1] + g2_ref[...]
    out_ref[...] = dis[:, None] * s + b_ref[...]


_DEG_SPEC = pl.BlockSpec((NC, BM, 16), lambda i: (0, i, 0))
_ROW_SPEC = pl.BlockSpec((BM, D), lambda i: (i, 0))
_ACC_SPEC = pl.BlockSpec((NC, BM, D), lambda i: (0, i, 0))
_W_SPEC = pl.BlockSpec((D, D), lambda i: (0, 0))
_B_SPEC = pl.BlockSpec((1, D), lambda i: (0, 0))
_OUT = jax.ShapeDtypeStruct((N, D), jnp.float32)

_tc_g1 = pl.pallas_call(
    _tc_g1_body, grid=(GRID,),
    in_specs=[_DEG_SPEC, _ROW_SPEC, _W_SPEC],
    out_specs=_ROW_SPEC, out_shape=_OUT)

_tc_mid = pl.pallas_call(
    _tc_mid_body, grid=(GRID,),
    in_specs=[_DEG_SPEC, _ACC_SPEC, _ROW_SPEC, _W_SPEC, _B_SPEC],
    out_specs=_ROW_SPEC, out_shape=_OUT)

_tc_fin = pl.pallas_call(
    _tc_fin_body, grid=(GRID,),
    in_specs=[_DEG_SPEC, _ACC_SPEC, _ROW_SPEC, _B_SPEC],
    out_specs=_ROW_SPEC, out_shape=_OUT)


def kernel(x, edge_index, W1, b1, W2, b2):
    src3 = edge_index[0].astype(jnp.int32).reshape(NW, NCHUNK, K)
    dst3 = edge_index[1].astype(jnp.int32).reshape(NW, NCHUNK, K)
    ones16 = jnp.ones((K, 16), jnp.float32)
    z16 = jnp.zeros((TROWS, 16), jnp.float32)
    z128 = jnp.zeros((ZROWS, D), jnp.float32)

    dstm3, degp = _sc_prep(src3, dst3, ones16, z16)
    degp3 = degp.reshape(NC, ACC_ROWS, 16)

    g1 = _tc_g1(degp3, x, W1)
    acc1 = _sc_agg(g1, src3, dstm3, z128).reshape(NC, ACC_ROWS, D)
    g2 = _tc_mid(degp3, acc1, g1, W2, b1.reshape(1, D))
    acc2 = _sc_agg(g2, src3, dstm3, z128).reshape(NC, ACC_ROWS, D)
    return _tc_fin(degp3, acc2, g2, b2.reshape(1, D))
